# baseline (speedup 1.0000x reference)
import jax
import jax.numpy as jnp
from jax.experimental import pallas as pl
from jax.experimental.pallas import tpu as pltpu

_BI = 400
_RC = 2
_CC = 2048


def _make_body(N, NPAD, BK):
    K = NPAD // BK

    def body(adj_ref, feat_ref, w1_ref, w2_ref, bias_ref,
             pre_ref, out_ref, acc_ref, cache_ref, g_ref, g16_ref, pre1_ref):
        p = pl.program_id(0)
        i = pl.program_id(1)
        k = pl.program_id(2)
        dn = (((1,), (1,)), ((), ()))

        def masked_adj():
            col = jax.lax.broadcasted_iota(jnp.int32, (_BI, BK), 1)
            return jnp.where(col < N - (K - 1) * BK, adj_ref[...], 0.0)

        @pl.when(p == 0)
        def _pass1():
            @pl.when((i == 0) & (k == 0))
            def _():
                z = jnp.zeros((NPAD - N, g_ref.shape[1]), jnp.float32)
                g_ref[pl.ds(N, NPAD - N), :] = z
                g16_ref[pl.ds(N, NPAD - N), :] = z.astype(jnp.bfloat16)

            @pl.when((i < _RC) & (k < K - 1))
            def _():
                for c in range(0, BK, _CC):
                    w = min(_CC, BK - c)
                    cache_ref[pl.ds(i * _BI, _BI), pl.ds(k * BK + c, w)] = (
                        adj_ref[:, pl.ds(c, w)].astype(jnp.bfloat16))

            @pl.when((i < _RC) & (k == K - 1))
            def _():
                a = masked_adj()
                for c in range(0, BK, _CC):
                    w = min(_CC, BK - c)
                    cache_ref[pl.ds(i * _BI, _BI), pl.ds(k * BK + c, w)] = (
                        a[:, c:c + w].astype(jnp.bfloat16))

            @pl.when(k < K - 1)
            def _():
                part = jnp.dot(adj_ref[...], feat_ref[pl.ds(k * BK, BK), :],
                               preferred_element_type=jnp.float32)

                @pl.when(k == 0)
                def _():
                    acc_ref[...] = part

                @pl.when(k > 0)
                def _():
                    acc_ref[...] += part

            @pl.when(k == K - 1)
            def _():
                ax = acc_ref[...] + jnp.dot(
                    masked_adj(), feat_ref[pl.ds((K - 1) * BK, BK), :],
                    preferred_element_type=jnp.float32)
                g = ax * feat_ref[pl.ds(i * _BI, _BI), :]
                g_ref[pl.ds(i * _BI, _BI), :] = g
                g16_ref[pl.ds(i * _BI, _BI), :] = g.astype(jnp.bfloat16)
                pre1_ref[pl.ds(i * _BI, _BI), :] = jax.lax.dot_general(
                    ax, w1_ref[...], dn,
                    precision=jax.lax.Precision.HIGHEST,
                    preferred_element_type=jnp.float32).astype(jnp.bfloat16)

        @pl.when(p == 1)
        def _pass2():
            @pl.when(i < _RC)
            def _():
                part = jnp.dot(
                    cache_ref[pl.ds(i * _BI, _BI), pl.ds(k * BK, BK)],
                    g16_ref[pl.ds(k * BK, BK), :],
                    preferred_element_type=jnp.float32)

                @pl.when(k == 0)
                def _():
                    acc_ref[...] = part

                @pl.when(k > 0)
                def _():
                    acc_ref[...] += part

            @pl.when((i >= _RC) & (k < K - 1))
            def _():
                part = jnp.dot(adj_ref[...], g_ref[pl.ds(k * BK, BK), :],
                               preferred_element_type=jnp.float32)

                @pl.when(k == 0)
                def _():
                    acc_ref[...] = part

                @pl.when(k > 0)
                def _():
                    acc_ref[...] += part

            @pl.when((i >= _RC) & (k == K - 1))
            def _():
                acc_ref[...] += jnp.dot(
                    masked_adj(), g_ref[pl.ds((K - 1) * BK, BK), :],
                    preferred_element_type=jnp.float32)

            @pl.when(k == K - 1)
            def _():
                pre = (
                    pre1_ref[pl.ds(i * _BI, _BI), :].astype(jnp.float32)
                    + jax.lax.dot_general(
                        acc_ref[...], w2_ref[...], dn,
                        precision=jax.lax.Precision.HIGHEST,
                        preferred_element_type=jnp.float32)
                    + bias_ref[...]
                )
                pre_ref[...] = pre
                out_ref[...] = jnp.where(pre > 0, pre, jnp.exp(pre) - 1.0)

    return body


def kernel(features, adj, W1, b1, W2, b2):
    N, H = features.shape
    BK = 5120 if N % 2048 else N // 2
    NPAD = 2 * BK
    R = N // _BI
    feat_pad = jnp.pad(features, ((0, NPAD - N), (0, 0)))
    bias = (b1 + b2).reshape(1, H)

    pre, out = pl.pallas_call(
        _make_body(N, NPAD, BK),
        grid=(2, R, 2),
        in_specs=[
            pl.BlockSpec(
                (_BI, BK),
                lambda p, i, k: (jnp.maximum(i, p * _RC),
                                 jnp.where((p == 1) & (i < _RC), 0, k))),
            pl.BlockSpec((NPAD, H), lambda p, i, k: (0, 0)),
            pl.BlockSpec((H, H), lambda p, i, k: (0, 0)),
            pl.BlockSpec((H, H), lambda p, i, k: (0, 0)),
            pl.BlockSpec((1, H), lambda p, i, k: (0, 0)),
        ],
        out_specs=[
            pl.BlockSpec((_BI, H), lambda p, i, k: (i * p, 0)),
            pl.BlockSpec((_BI, H), lambda p, i, k: (i * p, 0)),
        ],
        out_shape=[
            jax.ShapeDtypeStruct((N, H), jnp.float32),
            jax.ShapeDtypeStruct((N, H), jnp.float32),
        ],
        scratch_shapes=[
            pltpu.VMEM((_BI, H), jnp.float32),
            pltpu.VMEM((_RC * _BI, NPAD), jnp.bfloat16),
            pltpu.VMEM((NPAD, H), jnp.float32),
            pltpu.VMEM((NPAD, H), jnp.bfloat16),
            pltpu.VMEM((N, H), jnp.bfloat16),
        ],
    )(adj, feat_pad, W1, W2, bias)
    return (pre, out)


# speedup vs baseline: 1.0906x; 1.0906x over previous
import jax
import jax.numpy as jnp
from jax.experimental import pallas as pl
from jax.experimental.pallas import tpu as pltpu

_BI = 400
_RC = 2
_CC = 2048


def _body(adj_ref, feat_ref, w1_ref, w2_ref, bias_ref,
          pre_ref, out_ref, cache_ref, g16_ref, pre1_ref):
    p = pl.program_id(0)
    i = pl.program_id(1)
    N = feat_ref.shape[0]
    dn = (((1,), (1,)), ((), ()))

    @pl.when(p == 0)
    def _pass1():
        @pl.when(i < _RC)
        def _():
            for c in range(0, N, _CC):
                w = min(_CC, N - c)
                cache_ref[pl.ds(i * _BI, _BI), pl.ds(c, w)] = (
                    adj_ref[:, pl.ds(c, w)].astype(jnp.bfloat16))

        ax = jnp.dot(adj_ref[...], feat_ref[...],
                     preferred_element_type=jnp.float32)
        g = ax * feat_ref[pl.ds(i * _BI, _BI), :]
        g16_ref[pl.ds(i * _BI, _BI), :] = g.astype(jnp.bfloat16)
        pre1_ref[pl.ds(i * _BI, _BI), :] = jax.lax.dot_general(
            ax, w1_ref[...], dn,
            precision=jax.lax.Precision.HIGHEST,
            preferred_element_type=jnp.float32).astype(jnp.bfloat16)

    @pl.when(p == 1)
    def _pass2():
        def finish(axx):
            pre = (
                pre1_ref[pl.ds(i * _BI, _BI), :].astype(jnp.float32)
                + jax.lax.dot_general(
                    axx, w2_ref[...], dn,
                    precision=jax.lax.Precision.HIGHEST,
                    preferred_element_type=jnp.float32)
                + bias_ref[...]
            )
            pre_ref[...] = pre
            out_ref[...] = jnp.where(pre > 0, pre, jnp.exp(pre) - 1.0)

        @pl.when(i < _RC)
        def _():
            finish(jnp.dot(cache_ref[pl.ds(i * _BI, _BI), :], g16_ref[...],
                           preferred_element_type=jnp.float32))

        @pl.when(i >= _RC)
        def _():
            finish(jax.lax.dot_general(
                adj_ref[...], g16_ref[...], (((1,), (0,)), ((), ())),
                preferred_element_type=jnp.float32))


def kernel(features, adj, W1, b1, W2, b2):
    N, H = features.shape
    R = N // _BI
    bias = (b1 + b2).reshape(1, H)

    pre, out = pl.pallas_call(
        _body,
        grid=(2, R),
        in_specs=[
            pl.BlockSpec((_BI, N), lambda p, i: (jnp.maximum(i, p * _RC), 0)),
            pl.BlockSpec((N, H), lambda p, i: (0, 0)),
            pl.BlockSpec((H, H), lambda p, i: (0, 0)),
            pl.BlockSpec((H, H), lambda p, i: (0, 0)),
            pl.BlockSpec((1, H), lambda p, i: (0, 0)),
        ],
        out_specs=[
            pl.BlockSpec((_BI, H), lambda p, i: (i * p, 0)),
            pl.BlockSpec((_BI, H), lambda p, i: (i * p, 0)),
        ],
        out_shape=[
            jax.ShapeDtypeStruct((N, H), jnp.float32),
            jax.ShapeDtypeStruct((N, H), jnp.float32),
        ],
        scratch_shapes=[
            pltpu.VMEM((_RC * _BI, N), jnp.bfloat16),
            pltpu.VMEM((N, H), jnp.bfloat16),
            pltpu.VMEM((N, H), jnp.bfloat16),
        ],
    )(adj, features, W1, W2, bias)
    return (pre, out)
